# baseline (speedup 1.0000x reference)
import functools

import jax
import jax.numpy as jnp
from jax import lax
from jax.experimental import pallas as pl
from jax.experimental.pallas import tpu as pltpu
from jax.experimental.pallas import tpu_sc as plsc

_THRESHOLD = 0.0001
_LN2 = 0.6931471805599453
_LOG2C = (1.442687955e+00, -7.211279519e-01, 4.784177388e-01,
          -3.463918385e-01, 2.401828293e-01, -1.358065232e-01,
          5.115626768e-02, -9.118586525e-03)
_MIN_NORMAL = 1.1754943508222875e-38
_CH = 4
_NBUF = 3
_KB = 10


def _ln(x):
    xi = lax.bitcast_convert_type(x, jnp.int32)
    e = ((xi >> 23) & 0xFF) - 127
    m = lax.bitcast_convert_type((xi & 0x007FFFFF) | 0x3F800000, jnp.float32)
    t = m - 1.0
    acc = jnp.full_like(x, _LOG2C[-1])
    for c in _LOG2C[-2::-1]:
        acc = acc * t + c
    return (e.astype(jnp.float32) + acc * t) * _LN2


def _sc_body(out3, anc, tgt, part, buf, gt_v, tgt_v, anc_v, vals_v,
             gt_shared, sem, *, A, H, W, T, RPT, SB):
    c = lax.axis_index("c")
    s = lax.axis_index("s")
    w = c * 16 + s
    row0 = w * RPT
    npt = RPT * W
    nch = RPT // _CH
    zeros16f = jnp.zeros((16,), jnp.float32)
    lanes = lax.iota(jnp.int32, 16)
    zero16 = jnp.zeros((16,), jnp.int32)

    def _chunk_copy(k):
        return pltpu.make_async_copy(
            out3.at[pl.ds(row0 + k * _CH, _CH)],
            buf.at[k % _NBUF], sem.at[k % _NBUF])

    for kk in range(_NBUF - 1):
        _chunk_copy(kk).start()

    def _zero(i, _):
        gt_v[pl.ds(i * 16, 16)] = zeros16f
        return 0

    lax.fori_loop(0, npt // 16, _zero, 0)
    pltpu.sync_copy(gt_v, gt_shared.at[pl.ds(s * npt, npt)])
    plsc.subcore_barrier()

    @pl.when(s < SB)
    def _sparse():
        b = c * SB + s
        pltpu.sync_copy(tgt.at[b], tgt_v)
        pltpu.sync_copy(anc, anc_v)
        awh = []
        for k in range(A):
            aw = plsc.load_gather(anc_v, [zero16 + k, zero16])
            ah = plsc.load_gather(anc_v, [zero16 + k, zero16 + 1])
            awh.append((aw, ah))
        for k in range((T + 15) // 16):
            t_id = lanes + k * 16
            valid = t_id < T
            t_cl = jnp.minimum(t_id, T - 1)
            x = plsc.load_gather(tgt_v, [t_cl, zero16 + 1])
            y = plsc.load_gather(tgt_v, [t_cl, zero16 + 2])
            tw = plsc.load_gather(tgt_v, [t_cl, zero16 + 3]) * W
            th = plsc.load_gather(tgt_v, [t_cl, zero16 + 4]) * H
            t_i = jnp.clip((x * W).astype(jnp.int32), 0, W - 1)
            t_j = jnp.clip((y * H).astype(jnp.int32), 0, H - 1)
            t_area = tw * th
            ious = []
            for aw, ah in awh:
                inter = jnp.minimum(aw, tw) * jnp.minimum(ah, th)
                ious.append(inter / (aw * ah + t_area - inter))
            sel1 = (ious[1] > ious[0]) & (ious[1] >= ious[2])
            sel2 = (ious[2] > ious[0]) & (ious[2] > ious[1])
            iou_max = jnp.maximum(jnp.maximum(ious[0], ious[1]), ious[2])
            match = iou_max > _THRESHOLD
            a_vec = sel1.astype(jnp.int32) + 2 * sel2.astype(jnp.int32)
            flat = ((s * A + a_vec) * H + t_j) * W + t_i
            flat = jnp.where(valid, flat, 0)
            val = jnp.where(valid & match, 1.0, 0.0).astype(jnp.float32)
            vals_v[0, :] = val
            pltpu.sync_copy(vals_v.at[0], gt_shared.at[flat], add=True)

    plsc.subcore_barrier()

    pltpu.sync_copy(gt_shared.at[pl.ds(s * npt, npt)], gt_v)

    def _chunk(k, acc):
        @pl.when(k + _NBUF - 1 < nch)
        def _next():
            _chunk_copy(k + _NBUF - 1).start()

        _chunk_copy(k).wait()
        slot = k % _NBUF

        def _inner(i, acc):
            r = i >> 2
            cc = i & 3
            p = plsc.load_gather(
                buf, [zero16 + slot, zero16 + r, lanes + cc * 16, zero16 + 4])
            cnt = gt_v[pl.ds(k * (_CH * W) + i * 16, 16)]
            lp = jnp.maximum(_ln(jnp.maximum(p, _MIN_NORMAL)), -100.0)
            l1p = jnp.maximum(_ln(jnp.maximum(1.0 - p, _MIN_NORMAL)), -100.0)
            return acc + l1p + jnp.where(cnt > 0.5, lp - l1p, 0.0)

        return lax.fori_loop(0, (_CH * W) // 16, _inner, acc)

    acc = lax.fori_loop(0, nch, _chunk, zeros16f)
    vals_v[0, :] = acc
    pltpu.sync_copy(vals_v, part.at[w])


def _tc_body(out_ref, anc_ref, tgt_ref, sum_ref, acc_ref, *, A, H, W, KB):
    b = pl.program_id(0)

    @pl.when(b == 0)
    def _init():
        acc_ref[...] = jnp.zeros_like(acc_ref)

    for sb in range(2):
        _tc_batch(out_ref, anc_ref, tgt_ref, acc_ref, sb, A=A, H=H, W=W)

    @pl.when(b == pl.num_programs(0) - 1)
    def _fin():
        sum_ref[0, 0] = jnp.sum(acc_ref[...])


def _tc_batch(out_ref, anc_ref, tgt_ref, acc_ref, sb, *, A, H, W):
    tgt = tgt_ref[sb]
    x = tgt[:, 1:2]
    y = tgt[:, 2:3]
    tw = tgt[:, 3:4] * W
    th = tgt[:, 4:5] * H
    t_i = jnp.clip((x * W).astype(jnp.int32), 0, W - 1)
    t_j = jnp.clip((y * H).astype(jnp.int32), 0, H - 1)

    t_area = tw * th
    ious = []
    for k in range(A):
        aw = anc_ref[k, 0]
        ah = anc_ref[k, 1]
        inter = jnp.minimum(aw, tw) * jnp.minimum(ah, th)
        ious.append(inter / (aw * ah + t_area - inter))
    sel = [
        (ious[0] >= ious[1]) & (ious[0] >= ious[2]),
        (ious[1] > ious[0]) & (ious[1] >= ious[2]),
        (ious[2] > ious[0]) & (ious[2] > ious[1]),
    ]
    iou_max = jnp.maximum(jnp.maximum(ious[0], ious[1]), ious[2])
    match = iou_max > _THRESHOLD
    flags = [(match & s).astype(jnp.float32) for s in sel]

    T = tgt.shape[0]
    row_iota = lax.broadcasted_iota(jnp.int32, (T, H), 1)
    col_iota = lax.broadcasted_iota(jnp.int32, (T, W), 1)
    onehot_i = jnp.where(col_iota == t_i, 1.0, 0.0)
    contrib = acc_ref[...]
    for a in range(A):
        onehot_j = jnp.where(row_iota == t_j, flags[a], 0.0)
        counts = lax.dot_general(
            onehot_j, onehot_i,
            dimension_numbers=(((0,), (0,)), ((), ())),
            preferred_element_type=jnp.float32,
            precision=lax.Precision.HIGHEST,
        )
        gt = (counts > 0.5).astype(jnp.float32)
        pred = out_ref[sb, a, :, :, 4]
        lp = jnp.maximum(jnp.log(pred), -100.0)
        l1p = jnp.maximum(jnp.log(1.0 - pred), -100.0)
        contrib = contrib + (l1p + gt * (lp - l1p))
    acc_ref[...] = contrib


def _finish_body(part_ref, tcsum_ref, loss_ref, *, N):
    loss_ref[0, 0] = -(jnp.sum(part_ref[...]) + tcsum_ref[0, 0]) / N


def kernel(output, anchors, targets):
    B, A, H, W, C = output.shape
    T = targets.shape[1]
    N = B * A * H * W
    SC_ROWS = _KB * A * H
    RPT = SC_ROWS // 32
    SB = _KB // 2
    out3 = output.reshape(B * A * H, W, C)

    mesh = plsc.VectorSubcoreMesh(core_axis_name="c", subcore_axis_name="s")
    body = functools.partial(_sc_body, A=A, H=H, W=W, T=T, RPT=RPT, SB=SB)
    npt = RPT * W
    part = pl.kernel(
        body,
        out_type=jax.ShapeDtypeStruct((32, 1, 16), jnp.float32),
        mesh=mesh,
        scratch_types=[
            pltpu.VMEM((_NBUF, _CH, W, C), jnp.float32),
            pltpu.VMEM((npt,), jnp.float32),
            pltpu.VMEM((T, 5), jnp.float32),
            pltpu.VMEM((A, 2), jnp.float32),
            pltpu.VMEM((1, 16), jnp.float32),
            pltpu.VMEM_SHARED((16 * npt,), jnp.float32),
            pltpu.SemaphoreType.DMA((_NBUF,)),
        ],
        compiler_params=pltpu.CompilerParams(use_tc_tiling_on_sc=True,
                                             needs_layout_passes=False),
    )(out3, anchors, targets)

    if B > _KB:
        tc_sum = pl.pallas_call(
            functools.partial(_tc_body, A=A, H=H, W=W, KB=_KB),
            grid=((B - _KB) // 2,),
            in_specs=[
                pl.BlockSpec((2, A, H, W, C), lambda b: (b + _KB // 2, 0, 0, 0, 0)),
                pl.BlockSpec(memory_space=pltpu.SMEM),
                pl.BlockSpec((2, T, 5), lambda b: (b + _KB // 2, 0, 0)),
            ],
            out_specs=pl.BlockSpec(memory_space=pltpu.SMEM),
            out_shape=jax.ShapeDtypeStruct((1, 1), jnp.float32),
            scratch_shapes=[pltpu.VMEM((H, W), jnp.float32)],
        )(output, anchors, targets)
    else:
        tc_sum = jnp.zeros((1, 1), jnp.float32)

    loss2d = pl.pallas_call(
        functools.partial(_finish_body, N=float(N)),
        in_specs=[pl.BlockSpec((32, 1, 16), lambda: (0, 0, 0)),
                  pl.BlockSpec(memory_space=pltpu.SMEM)],
        out_specs=pl.BlockSpec(memory_space=pltpu.SMEM),
        out_shape=jax.ShapeDtypeStruct((1, 1), jnp.float32),
    )(part, tc_sum)
    return loss2d.reshape(())


# speedup vs baseline: 1.0841x; 1.0841x over previous
import functools

import jax
import jax.numpy as jnp
from jax import lax
from jax.experimental import pallas as pl
from jax.experimental.pallas import tpu as pltpu
from jax.experimental.pallas import tpu_sc as plsc

_THRESHOLD = 0.0001
_LN2 = 0.6931471805599453
_LOG2C = (1.442687955e+00, -7.211279519e-01, 4.784177388e-01,
          -3.463918385e-01, 2.401828293e-01, -1.358065232e-01,
          5.115626768e-02, -9.118586525e-03)
_MIN_NORMAL = 1.1754943508222875e-38
_CH = 4
_NBUF = 3
_KB = 10


def _ln(x):
    xi = lax.bitcast_convert_type(x, jnp.int32)
    e = ((xi >> 23) & 0xFF) - 127
    m = lax.bitcast_convert_type((xi & 0x007FFFFF) | 0x3F800000, jnp.float32)
    t = m - 1.0
    acc = jnp.full_like(x, _LOG2C[-1])
    for c in _LOG2C[-2::-1]:
        acc = acc * t + c
    return (e.astype(jnp.float32) + acc * t) * _LN2


def _sc_body(out3, anc, tgt, part, buf, gt_v, tgt_v, anc_v, vals_v,
             gt_shared, sem, *, A, H, W, T, RPT, SB):
    c = lax.axis_index("c")
    s = lax.axis_index("s")
    w = c * 16 + s
    row0 = w * RPT
    npt = RPT * W
    nch = RPT // _CH
    zeros16f = jnp.zeros((16,), jnp.float32)
    lanes = lax.iota(jnp.int32, 16)
    zero16 = jnp.zeros((16,), jnp.int32)

    def _chunk_copy(k):
        return pltpu.make_async_copy(
            out3.at[pl.ds(row0 + k * _CH, _CH)],
            buf.at[k % _NBUF], sem.at[k % _NBUF])

    for kk in range(_NBUF - 1):
        _chunk_copy(kk).start()

    def _zero(i, _):
        gt_v[pl.ds(i * 16, 16)] = zeros16f
        return 0

    lax.fori_loop(0, npt // 16, _zero, 0)
    pltpu.sync_copy(gt_v, gt_shared.at[pl.ds(s * npt, npt)])
    plsc.subcore_barrier()

    @pl.when(s < SB)
    def _sparse():
        b = c * SB + s
        pltpu.sync_copy(tgt.at[b], tgt_v)
        pltpu.sync_copy(anc, anc_v)
        awh = []
        for k in range(A):
            aw = plsc.load_gather(anc_v, [zero16 + k, zero16])
            ah = plsc.load_gather(anc_v, [zero16 + k, zero16 + 1])
            awh.append((aw, ah))
        for k in range((T + 15) // 16):
            t_id = lanes + k * 16
            valid = t_id < T
            t_cl = jnp.minimum(t_id, T - 1)
            x = plsc.load_gather(tgt_v, [t_cl, zero16 + 1])
            y = plsc.load_gather(tgt_v, [t_cl, zero16 + 2])
            tw = plsc.load_gather(tgt_v, [t_cl, zero16 + 3]) * W
            th = plsc.load_gather(tgt_v, [t_cl, zero16 + 4]) * H
            t_i = jnp.clip((x * W).astype(jnp.int32), 0, W - 1)
            t_j = jnp.clip((y * H).astype(jnp.int32), 0, H - 1)
            t_area = tw * th
            ious = []
            for aw, ah in awh:
                inter = jnp.minimum(aw, tw) * jnp.minimum(ah, th)
                ious.append(inter / (aw * ah + t_area - inter))
            sel1 = (ious[1] > ious[0]) & (ious[1] >= ious[2])
            sel2 = (ious[2] > ious[0]) & (ious[2] > ious[1])
            iou_max = jnp.maximum(jnp.maximum(ious[0], ious[1]), ious[2])
            match = iou_max > _THRESHOLD
            a_vec = sel1.astype(jnp.int32) + 2 * sel2.astype(jnp.int32)
            flat = ((s * A + a_vec) * H + t_j) * W + t_i
            flat = jnp.where(valid, flat, 0)
            val = jnp.where(valid & match, 1.0, 0.0).astype(jnp.float32)
            vals_v[0, :] = val
            pltpu.sync_copy(vals_v.at[0], gt_shared.at[flat], add=True)

    plsc.subcore_barrier()

    pltpu.sync_copy(gt_shared.at[pl.ds(s * npt, npt)], gt_v)

    def _chunk(k, acc):
        @pl.when(k + _NBUF - 1 < nch)
        def _next():
            _chunk_copy(k + _NBUF - 1).start()

        _chunk_copy(k).wait()
        slot = k % _NBUF

        def _inner(i, acc):
            r = i >> 2
            cc = i & 3
            p = plsc.load_gather(
                buf, [zero16 + slot, zero16 + r, lanes + cc * 16, zero16 + 4])
            cnt = gt_v[pl.ds(k * (_CH * W) + i * 16, 16)]
            lp = jnp.maximum(_ln(jnp.maximum(p, _MIN_NORMAL)), -100.0)
            l1p = jnp.maximum(_ln(jnp.maximum(1.0 - p, _MIN_NORMAL)), -100.0)
            return acc + l1p + jnp.where(cnt > 0.5, lp - l1p, 0.0)

        return lax.fori_loop(0, (_CH * W) // 16, _inner, acc)

    acc = lax.fori_loop(0, nch, _chunk, zeros16f)
    vals_v[0, :] = acc
    pltpu.sync_copy(vals_v, part.at[w])


def _tc_body(out_ref, anc_ref, tgt_ref, sum_ref, acc_ref, *, A, H, W, KB):
    b = pl.program_id(0)

    @pl.when(b == 0)
    def _init():
        acc_ref[...] = jnp.zeros_like(acc_ref)

    _tc_batch(out_ref, anc_ref, tgt_ref, acc_ref, 0, A=A, H=H, W=W)

    @pl.when(b == pl.num_programs(0) - 1)
    def _fin():
        sum_ref[0, 0] = jnp.sum(acc_ref[...])


def _tc_batch(out_ref, anc_ref, tgt_ref, acc_ref, sb, *, A, H, W):
    tgt = tgt_ref[sb]
    x = tgt[:, 1:2]
    y = tgt[:, 2:3]
    tw = tgt[:, 3:4] * W
    th = tgt[:, 4:5] * H
    t_i = jnp.clip((x * W).astype(jnp.int32), 0, W - 1)
    t_j = jnp.clip((y * H).astype(jnp.int32), 0, H - 1)

    t_area = tw * th
    ious = []
    for k in range(A):
        aw = anc_ref[k, 0]
        ah = anc_ref[k, 1]
        inter = jnp.minimum(aw, tw) * jnp.minimum(ah, th)
        ious.append(inter / (aw * ah + t_area - inter))
    sel = [
        (ious[0] >= ious[1]) & (ious[0] >= ious[2]),
        (ious[1] > ious[0]) & (ious[1] >= ious[2]),
        (ious[2] > ious[0]) & (ious[2] > ious[1]),
    ]
    iou_max = jnp.maximum(jnp.maximum(ious[0], ious[1]), ious[2])
    match = iou_max > _THRESHOLD
    flags = [(match & s).astype(jnp.float32) for s in sel]

    T = tgt.shape[0]
    row_iota = lax.broadcasted_iota(jnp.int32, (T, H), 1)
    col_iota = lax.broadcasted_iota(jnp.int32, (T, W), 1)
    onehot_i = jnp.where(col_iota == t_i, 1.0, 0.0)
    contrib = acc_ref[...]
    for a in range(A):
        onehot_j = jnp.where(row_iota == t_j, flags[a], 0.0)
        counts = lax.dot_general(
            onehot_j, onehot_i,
            dimension_numbers=(((0,), (0,)), ((), ())),
            preferred_element_type=jnp.float32,
            precision=lax.Precision.HIGHEST,
        )
        gt = (counts > 0.5).astype(jnp.float32)
        pred = out_ref[sb, a, :, :, 4]
        lp = jnp.maximum(jnp.log(pred), -100.0)
        l1p = jnp.maximum(jnp.log(1.0 - pred), -100.0)
        contrib = contrib + (l1p + gt * (lp - l1p))
    acc_ref[...] = contrib


def _finish_body(part_ref, tcsum_ref, loss_ref, *, N):
    loss_ref[0, 0] = -(jnp.sum(part_ref[...]) + tcsum_ref[0, 0]) / N


def kernel(output, anchors, targets):
    B, A, H, W, C = output.shape
    T = targets.shape[1]
    N = B * A * H * W
    SC_ROWS = _KB * A * H
    RPT = SC_ROWS // 32
    SB = _KB // 2
    out3 = output.reshape(B * A * H, W, C)

    mesh = plsc.VectorSubcoreMesh(core_axis_name="c", subcore_axis_name="s")
    body = functools.partial(_sc_body, A=A, H=H, W=W, T=T, RPT=RPT, SB=SB)
    npt = RPT * W
    part = pl.kernel(
        body,
        out_type=jax.ShapeDtypeStruct((32, 1, 16), jnp.float32),
        mesh=mesh,
        scratch_types=[
            pltpu.VMEM((_NBUF, _CH, W, C), jnp.float32),
            pltpu.VMEM((npt,), jnp.float32),
            pltpu.VMEM((T, 5), jnp.float32),
            pltpu.VMEM((A, 2), jnp.float32),
            pltpu.VMEM((1, 16), jnp.float32),
            pltpu.VMEM_SHARED((16 * npt,), jnp.float32),
            pltpu.SemaphoreType.DMA((_NBUF,)),
        ],
        compiler_params=pltpu.CompilerParams(use_tc_tiling_on_sc=True,
                                             needs_layout_passes=False),
    )(out3, anchors, targets)

    if B > _KB:
        tc_sum = pl.pallas_call(
            functools.partial(_tc_body, A=A, H=H, W=W, KB=_KB),
            grid=(B - _KB,),
            in_specs=[
                pl.BlockSpec((1, A, H, W, C), lambda b: (b + _KB, 0, 0, 0, 0)),
                pl.BlockSpec(memory_space=pltpu.SMEM),
                pl.BlockSpec((1, T, 5), lambda b: (b + _KB, 0, 0)),
            ],
            out_specs=pl.BlockSpec(memory_space=pltpu.SMEM),
            out_shape=jax.ShapeDtypeStruct((1, 1), jnp.float32),
            scratch_shapes=[pltpu.VMEM((H, W), jnp.float32)],
        )(output, anchors, targets)
    else:
        tc_sum = jnp.zeros((1, 1), jnp.float32)

    loss2d = pl.pallas_call(
        functools.partial(_finish_body, N=float(N)),
        in_specs=[pl.BlockSpec((32, 1, 16), lambda: (0, 0, 0)),
                  pl.BlockSpec(memory_space=pltpu.SMEM)],
        out_specs=pl.BlockSpec(memory_space=pltpu.SMEM),
        out_shape=jax.ShapeDtypeStruct((1, 1), jnp.float32),
    )(part, tc_sum)
    return loss2d.reshape(())
